# baseline (speedup 1.0000x reference)
import functools

import jax
import jax.numpy as jnp
from jax import lax
from jax.experimental import pallas as pl
from jax.experimental.pallas import tpu as pltpu
from jax.experimental.pallas import tpu_sc as plsc

NC = 2
NS = 16
NW = NC * NS
LANES = 16


def _tc_repack(emb):
    V, d = emb.shape
    BS = 10000
    assert V % BS == 0

    def body(i_ref, o_ref):
        o_ref[:, :d] = i_ref[...]

    return pl.pallas_call(
        body,
        grid=(V // BS,),
        in_specs=[pl.BlockSpec((BS, d), lambda i: (i, 0))],
        out_specs=pl.BlockSpec((BS, 2 * d), lambda i: (i, 0)),
        out_shape=jax.ShapeDtypeStruct((V, 2 * d), jnp.float32),
    )(emb)


def _sc_bag_sums(text2d, emb_weight, *, n_chunk_rows, chunk_tok, d, chunk_bags, hist):
    chunks_per_w = n_chunk_rows // NW
    bags_per_w = chunks_per_w * chunk_bags
    out_elems_per_w = bags_per_w * d
    nbuf = 4
    assert chunks_per_w % nbuf == 0

    mesh = plsc.VectorSubcoreMesh(core_axis_name="c", subcore_axis_name="s")

    @functools.partial(
        pl.kernel,
        mesh=mesh,
        compiler_params=pltpu.CompilerParams(use_tc_tiling_on_sc=False),
        out_type=jax.ShapeDtypeStruct((n_chunk_rows * chunk_bags * d,), jnp.float32),
        scratch_types=[
            pltpu.VMEM((chunks_per_w, chunk_tok), jnp.int32),
            pltpu.VMEM((out_elems_per_w,), jnp.float32),
        ]
        + [pltpu.VMEM((chunk_tok, 2 * d), jnp.float32) for _ in range(nbuf)]
        + [pltpu.SemaphoreType.DMA for _ in range(nbuf)],
    )
    def body(text_hbm, table_hbm, out_hbm, idx_v, sums_v, *bufs_sems):
        bufs, sems = bufs_sems[:nbuf], bufs_sems[nbuf:]
        wid = lax.axis_index("s") * NC + lax.axis_index("c")
        pltpu.sync_copy(text_hbm.at[pl.ds(wid * chunks_per_w, chunks_per_w)], idx_v)

        def gather(c, b):
            pltpu.async_copy(table_hbm.at[idx_v.at[c]], bufs[b], sems[b])

        for b in range(nbuf):
            gather(b, b)

        def group_body(i, carry):
            c0 = i * nbuf
            for b in range(nbuf):
                c = c0 + b
                pltpu.make_async_copy(
                    table_hbm.at[idx_v.at[c]], bufs[b], sems[b]).wait()
                for j in range(chunk_bags):
                    for g in range(d // LANES):
                        acc = bufs[b][j * hist, g * LANES:(g + 1) * LANES]
                        for t in range(1, hist):
                            acc = acc + bufs[b][j * hist + t, g * LANES:(g + 1) * LANES]
                        base = (c * chunk_bags + j) * d + g * LANES
                        sums_v[pl.ds(base, LANES)] = acc

                @pl.when(c + nbuf < chunks_per_w)
                def _():
                    gather(c + nbuf, b)
            return carry

        lax.fori_loop(0, chunks_per_w // nbuf, group_body, 0)
        pltpu.sync_copy(sums_v, out_hbm.at[pl.ds(wid * out_elems_per_w, out_elems_per_w)])

    return body(text2d, emb_weight)


def _tc_fc(sums2d, w_pad, b_pad):
    def fc_body(s_ref, w_ref, b_ref, o_ref):
        o_ref[...] = (
            jnp.dot(s_ref[...], w_ref[...], preferred_element_type=jnp.float32)
            + b_ref[...]
        )

    return pl.pallas_call(
        fc_body,
        out_shape=jax.ShapeDtypeStruct((sums2d.shape[0], w_pad.shape[1]), jnp.float32),
    )(sums2d, w_pad, b_pad)


def kernel(text, offsets, emb_weight, fc_w, fc_b):
    T = text.shape[0]
    B = offsets.shape[0]
    hist = T // B
    d = emb_weight.shape[1]
    nclass = fc_w.shape[0]

    chunk_bags = 2
    chunk_tok = chunk_bags * hist
    n_chunk_rows = B // chunk_bags

    text2d = text.astype(jnp.int32).reshape(n_chunk_rows, chunk_tok)
    table128 = _tc_repack(emb_weight)
    sums_flat = _sc_bag_sums(
        text2d, table128,
        n_chunk_rows=n_chunk_rows, chunk_tok=chunk_tok, d=d,
        chunk_bags=chunk_bags, hist=hist,
    )
    sums2d = sums_flat.reshape(B, d)

    w_pad = jnp.zeros((d, 128), jnp.float32).at[:, :nclass].set(fc_w.T / float(hist))
    b_pad = jnp.zeros((1, 128), jnp.float32).at[0, :nclass].set(fc_b)
    out = _tc_fc(sums2d, w_pad, b_pad)
    return out[:, :nclass]


# speedup vs baseline: 1.0005x; 1.0005x over previous
import functools

import jax
import jax.numpy as jnp
from jax import lax
from jax.experimental import pallas as pl
from jax.experimental.pallas import tpu as pltpu
from jax.experimental.pallas import tpu_sc as plsc

NC = 2
NS = 16
NW = NC * NS
LANES = 16


def _tc_repack(emb):
    V, d = emb.shape
    BS = 10000
    assert V % BS == 0

    def body(i_ref, o_ref):
        o_ref[:, :d] = i_ref[...]

    return pl.pallas_call(
        body,
        grid=(V // BS,),
        in_specs=[pl.BlockSpec((BS, d), lambda i: (i, 0))],
        out_specs=pl.BlockSpec((BS, 2 * d), lambda i: (i, 0)),
        out_shape=jax.ShapeDtypeStruct((V, 2 * d), jnp.float32),
    )(emb)


def _sc_bag_sums(text2d, table128, *, n_chunk_rows, chunk_tok, d, chunk_bags, hist):
    chunks_per_w = n_chunk_rows // NW
    bags_per_w = chunks_per_w * chunk_bags
    out_elems_per_w = bags_per_w * d
    nbuf = 4
    assert chunks_per_w % nbuf == 0

    mesh = plsc.VectorSubcoreMesh(core_axis_name="c", subcore_axis_name="s")

    @functools.partial(
        pl.kernel,
        mesh=mesh,
        compiler_params=pltpu.CompilerParams(use_tc_tiling_on_sc=True),
        out_type=jax.ShapeDtypeStruct((n_chunk_rows * chunk_bags * d,), jnp.float32),
        scratch_types=[
            pltpu.VMEM((chunks_per_w, 128), jnp.int32),
            pltpu.VMEM((out_elems_per_w,), jnp.float32),
        ]
        + [pltpu.VMEM((chunk_tok, 2 * d), jnp.float32) for _ in range(nbuf)]
        + [pltpu.SemaphoreType.DMA for _ in range(nbuf)],
    )
    def body(text_hbm, table_hbm, out_hbm, idx_v, sums_v, *bufs_sems):
        bufs, sems = bufs_sems[:nbuf], bufs_sems[nbuf:]
        wid = lax.axis_index("s") * NC + lax.axis_index("c")
        pltpu.sync_copy(text_hbm.at[pl.ds(wid * chunks_per_w, chunks_per_w)], idx_v)

        def gather(c, b):
            pltpu.async_copy(
                table_hbm.at[idx_v.at[c, pl.ds(0, chunk_tok)]], bufs[b], sems[b])

        for b in range(nbuf):
            gather(b, b)

        def group_body(i, carry):
            c0 = i * nbuf
            for b in range(nbuf):
                c = c0 + b
                pltpu.make_async_copy(
                    table_hbm.at[idx_v.at[c, pl.ds(0, chunk_tok)]],
                    bufs[b], sems[b]).wait()
                for j in range(chunk_bags):
                    for g in range(d // LANES):
                        acc = bufs[b][j * hist, g * LANES:(g + 1) * LANES]
                        for t in range(1, hist):
                            acc = acc + bufs[b][j * hist + t, g * LANES:(g + 1) * LANES]
                        base = (c * chunk_bags + j) * d + g * LANES
                        sums_v[pl.ds(base, LANES)] = acc

                @pl.when(c + nbuf < chunks_per_w)
                def _():
                    gather(c + nbuf, b)
            return carry

        lax.fori_loop(0, chunks_per_w // nbuf, group_body, 0)
        pltpu.sync_copy(sums_v, out_hbm.at[pl.ds(wid * out_elems_per_w, out_elems_per_w)])

    return body(text2d, table128)


def _tc_fc(sums2d, w_pad, b_pad):
    def fc_body(s_ref, w_ref, b_ref, o_ref):
        o_ref[...] = (
            jnp.dot(s_ref[...], w_ref[...], preferred_element_type=jnp.float32)
            + b_ref[...]
        )

    return pl.pallas_call(
        fc_body,
        out_shape=jax.ShapeDtypeStruct((sums2d.shape[0], w_pad.shape[1]), jnp.float32),
    )(sums2d, w_pad, b_pad)


def kernel(text, offsets, emb_weight, fc_w, fc_b):
    T = text.shape[0]
    B = offsets.shape[0]
    hist = T // B
    d = emb_weight.shape[1]
    nclass = fc_w.shape[0]

    chunk_bags = 2
    chunk_tok = chunk_bags * hist
    n_chunk_rows = B // chunk_bags

    text2d = text.astype(jnp.int32).reshape(n_chunk_rows, chunk_tok)
    text2d = jnp.pad(text2d, ((0, 0), (0, 128 - chunk_tok)))
    table128 = _tc_repack(emb_weight)
    sums_flat = _sc_bag_sums(
        text2d, table128,
        n_chunk_rows=n_chunk_rows, chunk_tok=chunk_tok, d=d,
        chunk_bags=chunk_bags, hist=hist,
    )
    sums2d = sums_flat.reshape(B, d)

    w_pad = jnp.zeros((d, 128), jnp.float32).at[:, :nclass].set(fc_w.T / float(hist))
    b_pad = jnp.zeros((1, 128), jnp.float32).at[0, :nclass].set(fc_b)
    out = _tc_fc(sums2d, w_pad, b_pad)
    return out[:, :nclass]


# speedup vs baseline: 2.1871x; 2.1861x over previous
import functools

import jax
import jax.numpy as jnp
from jax import lax
from jax.experimental import pallas as pl
from jax.experimental.pallas import tpu as pltpu
from jax.experimental.pallas import tpu_sc as plsc

NC = 2
NS = 16
NW = NC * NS
LANES = 16
BS = 16384


def _tc_repack(emb, H):
    d, V = emb.shape
    nblk = H // BS

    def body(lo_ref, hi_ref, o_ref):
        o_ref[:, :d] = lo_ref[...].T
        o_ref[:, d:] = hi_ref[...].T

    return pl.pallas_call(
        body,
        grid=(nblk,),
        in_specs=[
            pl.BlockSpec((d, BS), lambda i: (0, i)),
            pl.BlockSpec((d, BS), lambda i, _n=nblk: (0, _n + i)),
        ],
        out_specs=pl.BlockSpec((BS, 2 * d), lambda i: (i, 0)),
        out_shape=jax.ShapeDtypeStruct((H, 2 * d), jnp.float32),
    )(emb, emb)


def _sc_bag_sums(idx2d, off2d, table, *, n_chunk_rows, chunk_tok, d, chunk_bags, hist):
    chunks_per_w = n_chunk_rows // NW
    bags_per_w = chunks_per_w * chunk_bags
    out_elems_per_w = bags_per_w * d
    nbuf = 4
    assert chunks_per_w % nbuf == 0

    mesh = plsc.VectorSubcoreMesh(core_axis_name="c", subcore_axis_name="s")

    @functools.partial(
        pl.kernel,
        mesh=mesh,
        compiler_params=pltpu.CompilerParams(
            use_tc_tiling_on_sc=True, needs_layout_passes=False),
        out_type=jax.ShapeDtypeStruct((n_chunk_rows * chunk_bags * d,), jnp.float32),
        scratch_types=[
            pltpu.VMEM((chunks_per_w, 128), jnp.int32),
            pltpu.VMEM((chunks_per_w, 128), jnp.int32),
            pltpu.VMEM((out_elems_per_w,), jnp.float32),
        ]
        + [pltpu.VMEM((chunk_tok, 2 * d), jnp.float32) for _ in range(nbuf)]
        + [pltpu.SemaphoreType.DMA for _ in range(nbuf)],
    )
    def body(idx_hbm, off_hbm, table_hbm, out_hbm, idx_v, off_v, sums_v, *bufs_sems):
        bufs, sems = bufs_sems[:nbuf], bufs_sems[nbuf:]
        wid = lax.axis_index("s") * NC + lax.axis_index("c")
        pltpu.sync_copy(idx_hbm.at[pl.ds(wid * chunks_per_w, chunks_per_w)], idx_v)
        pltpu.sync_copy(off_hbm.at[pl.ds(wid * chunks_per_w, chunks_per_w)], off_v)

        iota = jnp.arange(LANES, dtype=jnp.int32)

        def gather(c, b):
            pltpu.async_copy(
                table_hbm.at[idx_v.at[c, pl.ds(0, chunk_tok)]], bufs[b], sems[b])

        for b in range(nbuf):
            gather(b, b)

        def group_body(i, carry):
            c0 = i * nbuf
            for b in range(nbuf):
                c = c0 + b
                pltpu.make_async_copy(
                    table_hbm.at[idx_v.at[c, pl.ds(0, chunk_tok)]],
                    bufs[b], sems[b]).wait()
                c_vec = jnp.zeros((LANES,), jnp.int32) + c
                for j in range(chunk_bags):
                    accs = [None] * (d // LANES)
                    for t in range(hist):
                        s = j * hist + t
                        s_vec = jnp.full((LANES,), s, jnp.int32)
                        off = plsc.load_gather(off_v, [c_vec, s_vec])
                        for g in range(d // LANES):
                            val = plsc.load_gather(
                                bufs[b], [s_vec, off + (iota + g * LANES)])
                            accs[g] = val if accs[g] is None else accs[g] + val
                    for g in range(d // LANES):
                        base = (c * chunk_bags + j) * d + g * LANES
                        sums_v[pl.ds(base, LANES)] = accs[g]

                @pl.when(c + nbuf < chunks_per_w)
                def _():
                    gather(c + nbuf, b)
            return carry

        lax.fori_loop(0, chunks_per_w // nbuf, group_body, 0)
        pltpu.sync_copy(sums_v, out_hbm.at[pl.ds(wid * out_elems_per_w, out_elems_per_w)])

    return body(idx2d, off2d, table)


def _tc_fc(sums2d, w_pad, b_pad):
    def fc_body(s_ref, w_ref, b_ref, o_ref):
        o_ref[...] = (
            jnp.dot(s_ref[...], w_ref[...], preferred_element_type=jnp.float32)
            + b_ref[...]
        )

    return pl.pallas_call(
        fc_body,
        out_shape=jax.ShapeDtypeStruct((sums2d.shape[0], w_pad.shape[1]), jnp.float32),
    )(sums2d, w_pad, b_pad)


def kernel(text, offsets, emb_weight, fc_w, fc_b):
    T = text.shape[0]
    B = offsets.shape[0]
    hist = T // B
    V, d = emb_weight.shape
    nclass = fc_w.shape[0]

    chunk_bags = 2
    chunk_tok = chunk_bags * hist
    n_chunk_rows = B // chunk_bags
    H = -(-(V // 2) // BS) * BS

    tok = text.astype(jnp.int32)
    hi = tok >= H
    idx2d = jnp.where(hi, tok - H, tok).reshape(n_chunk_rows, chunk_tok)
    off2d = jnp.where(hi, d, 0).astype(jnp.int32).reshape(n_chunk_rows, chunk_tok)
    idx2d = jnp.pad(idx2d, ((0, 0), (0, 128 - chunk_tok)))
    off2d = jnp.pad(off2d, ((0, 0), (0, 128 - chunk_tok)))

    table = _tc_repack(emb_weight.T, H)
    sums_flat = _sc_bag_sums(
        idx2d, off2d, table,
        n_chunk_rows=n_chunk_rows, chunk_tok=chunk_tok, d=d,
        chunk_bags=chunk_bags, hist=hist,
    )
    sums2d = sums_flat.reshape(B, d)

    w_pad = jnp.zeros((d, 128), jnp.float32).at[:, :nclass].set(fc_w.T / float(hist))
    b_pad = jnp.zeros((1, 128), jnp.float32).at[0, :nclass].set(fc_b)
    out = _tc_fc(sums2d, w_pad, b_pad)
    return out[:, :nclass]
